# Initial kernel scaffold; baseline (speedup 1.0000x reference)
import jax
import jax.numpy as jnp
from jax.experimental import pallas as pl


def kernel(x, emb_table):
    raise NotImplementedError("write your pallas kernel here")


# speedup vs baseline: 1.9759x; 1.9759x over previous
import jax
import jax.numpy as jnp
from jax.experimental import pallas as pl

BATCH = 4
SEQ = 2048
DM = 1024
BS = 512


def _add_body(x_ref, emb_ref, o_ref):
    off = (pl.program_id(0) * BS) % SEQ
    o_ref[:, :] = x_ref[:, :] + emb_ref[pl.ds(off, BS), :]


def kernel(x, emb_table):
    xf = x.reshape(BATCH * SEQ, DM)
    out = pl.pallas_call(
        _add_body,
        grid=(BATCH * SEQ // BS,),
        in_specs=[
            pl.BlockSpec((BS, DM), lambda i: (i, 0)),
            pl.BlockSpec((SEQ, DM), lambda i: (0, 0)),
        ],
        out_specs=pl.BlockSpec((BS, DM), lambda i: (i, 0)),
        out_shape=jax.ShapeDtypeStruct((BATCH * SEQ, DM), x.dtype),
    )(xf, emb_table)
    return out.reshape(BATCH, SEQ, DM)


# speedup vs baseline: 2.1466x; 1.0864x over previous
import jax
import jax.numpy as jnp
from jax.experimental import pallas as pl

BATCH = 4
SEQ = 2048
DM = 1024
BS = 1024


def _add_body(x_ref, emb_ref, o_ref):
    off = (pl.program_id(0) * BS) % SEQ
    o_ref[:, :] = x_ref[:, :] + emb_ref[pl.ds(off, BS), :]


def kernel(x, emb_table):
    xf = x.reshape(BATCH * SEQ, DM)
    out = pl.pallas_call(
        _add_body,
        grid=(BATCH * SEQ // BS,),
        in_specs=[
            pl.BlockSpec((BS, DM), lambda i: (i, 0)),
            pl.BlockSpec((SEQ, DM), lambda i: (0, 0)),
        ],
        out_specs=pl.BlockSpec((BS, DM), lambda i: (i, 0)),
        out_shape=jax.ShapeDtypeStruct((BATCH * SEQ, DM), x.dtype),
    )(xf, emb_table)
    return out.reshape(BATCH, SEQ, DM)


# speedup vs baseline: 2.2772x; 1.0608x over previous
import jax
import jax.numpy as jnp
from jax.experimental import pallas as pl

BATCH = 4
SEQ = 2048
DM = 1024
BS = 2048


def _add_body(x_ref, emb_ref, o_ref):
    off = (pl.program_id(0) * BS) % SEQ
    o_ref[:, :] = x_ref[:, :] + emb_ref[pl.ds(off, BS), :]


def kernel(x, emb_table):
    xf = x.reshape(BATCH * SEQ, DM)
    out = pl.pallas_call(
        _add_body,
        grid=(BATCH * SEQ // BS,),
        in_specs=[
            pl.BlockSpec((BS, DM), lambda i: (i, 0)),
            pl.BlockSpec((SEQ, DM), lambda i: (0, 0)),
        ],
        out_specs=pl.BlockSpec((BS, DM), lambda i: (i, 0)),
        out_shape=jax.ShapeDtypeStruct((BATCH * SEQ, DM), x.dtype),
    )(xf, emb_table)
    return out.reshape(BATCH, SEQ, DM)
